# baseline (speedup 1.0000x reference)
import functools

import jax
import jax.numpy as jnp
from jax import lax
from jax.experimental import pallas as pl
from jax.experimental.pallas import tpu as pltpu
from jax.experimental.pallas import tpu_sc as plsc

_BATCH = 4096
_FIELDS = 26
_EMB_DIM = 32
_VOCAB = 1000002
_CB = 512
_NBLK = (_VOCAB + _CB - 1) // _CB
_PROWS = _NBLK * 128
_TAIL = _VOCAB - 128
_TAIL_ROW = (_VOCAB // 128) * 32
_NC, _NS = 2, 16
_NW = _NC * _NS
_NB = _BATCH // 128
_TILES = _FIELDS * _NB
_TPW = _TILES // _NW

_mesh = plsc.VectorSubcoreMesh(core_axis_name="c", subcore_axis_name="s")
_params = pltpu.CompilerParams(needs_layout_passes=False)


def _tc_repack_body(tt_ref, out_ref):
    x = tt_ref[...]
    y = x.reshape(_EMB_DIM, 128, 4).transpose(1, 2, 0)
    out_ref[...] = y.reshape(128, 128)


_tc_repack = pl.pallas_call(
    _tc_repack_body,
    grid=(_NBLK,),
    in_specs=[pl.BlockSpec((_EMB_DIM, _CB), lambda m: (0, m))],
    out_specs=pl.BlockSpec((128, 128), lambda m: (m, 0)),
    out_shape=jax.ShapeDtypeStruct((_PROWS, 128), jnp.float32),
)


@functools.partial(
    pl.kernel,
    out_type=jax.ShapeDtypeStruct((_FIELDS, _EMB_DIM, _BATCH), jnp.float32),
    mesh=_mesh,
    scratch_types=[
        pltpu.VMEM((128,), jnp.int32),
        pltpu.VMEM((128,), jnp.int32),
        pltpu.VMEM((128,), jnp.int32),
        pltpu.VMEM((128, 136), jnp.float32),
        pltpu.VMEM((_EMB_DIM, 128), jnp.float32),
        pltpu.SemaphoreType.DMA,
    ],
    compiler_params=_params,
)
def _sc_gather(tpk_hbm, xt_hbm, out_hbm, idx_v, idx4_v, bcol_v, rows_v, out_v, sem):
    wid = lax.axis_index("s") * _NC + lax.axis_index("c")
    liota = lax.iota(jnp.int32, 16)

    def tile_body(t, carry):
        g = wid * _TPW + t
        f = g // _NB
        b = g % _NB
        pltpu.sync_copy(xt_hbm.at[f, pl.ds(b * 128, 128)], idx_v)

        def prep(q, c):
            v = idx_v[pl.ds(q * 16, 16)]
            idx4_v[pl.ds(q * 16, 16)] = lax.shift_right_logical(v, 2)
            bcol_v[pl.ds(q * 16, 16)] = lax.shift_left(
                lax.bitwise_and(v, 3), 5
            )
            return c

        lax.fori_loop(0, 8, prep, 0)
        pltpu.async_copy(
            tpk_hbm.at[idx4_v], rows_v.at[:, pl.ds(0, 128)], sem
        ).wait()

        def dq(q, c):
            lanes = liota + q * 16
            cols = bcol_v[pl.ds(q * 16, 16)]
            for dg in range(_EMB_DIM // 8):
                loads = [
                    plsc.load_gather(rows_v, [lanes, cols + (8 * dg + k)])
                    for k in range(8)
                ]
                for k in range(8):
                    out_v[8 * dg + k, pl.ds(q * 16, 16)] = loads[k]
            return c

        lax.fori_loop(0, 8, dq, 0)
        pltpu.sync_copy(out_v, out_hbm.at[f, :, pl.ds(b * 128, 128)])
        return carry

    lax.fori_loop(0, _TPW, tile_body, 0)


def kernel(x, filters, table):
    tt = table.T
    xt = x.T
    tpk = _tc_repack(tt)
    out_t = _sc_gather(tpk, xt)
    return (out_t.transpose(2, 0, 1), filters)


# speedup vs baseline: 4.7800x; 4.7800x over previous
import functools

import jax
import jax.numpy as jnp
from jax import lax
from jax.experimental import pallas as pl
from jax.experimental.pallas import tpu as pltpu
from jax.experimental.pallas import tpu_sc as plsc

_BATCH = 4096
_FIELDS = 26
_EMB_DIM = 32
_VOCAB = 1000002
_NC, _NS = 2, 16
_NW = _NC * _NS
_NB = _BATCH // 128
_TILES = _FIELDS * _NB
_TPW = _TILES // _NW

_mesh = plsc.VectorSubcoreMesh(core_axis_name="c", subcore_axis_name="s")
_params = pltpu.CompilerParams(
    needs_layout_passes=False, use_tc_tiling_on_sc=False
)


@functools.partial(
    pl.kernel,
    out_type=jax.ShapeDtypeStruct((_FIELDS, _EMB_DIM, _BATCH), jnp.float32),
    mesh=_mesh,
    scratch_types=[
        pltpu.VMEM((128,), jnp.int32),
        pltpu.VMEM((128, _EMB_DIM), jnp.float32),
        pltpu.VMEM((_EMB_DIM, 128), jnp.float32),
        pltpu.SemaphoreType.DMA,
    ],
    compiler_params=_params,
)
def _sc_gather(table_hbm, xt_hbm, out_hbm, idx_v, rows_v, out_v, sem):
    wid = lax.axis_index("s") * _NC + lax.axis_index("c")
    liota = lax.iota(jnp.int32, 16)

    def tile_body(t, carry):
        g = wid * _TPW + t
        f = g // _NB
        b = g % _NB
        pltpu.sync_copy(xt_hbm.at[f, pl.ds(b * 128, 128)], idx_v)
        pltpu.async_copy(table_hbm.at[idx_v], rows_v, sem).wait()

        def dq(q, c):
            lanes = liota + q * 16
            for dg in range(_EMB_DIM // 8):
                loads = [
                    plsc.load_gather(
                        rows_v,
                        [lanes, jnp.full((16,), 8 * dg + k, jnp.int32)],
                    )
                    for k in range(8)
                ]
                for k in range(8):
                    out_v[8 * dg + k, pl.ds(q * 16, 16)] = loads[k]
            return c

        lax.fori_loop(0, 8, dq, 0)
        pltpu.sync_copy(out_v, out_hbm.at[f, :, pl.ds(b * 128, 128)])
        return carry

    lax.fori_loop(0, _TPW, tile_body, 0)


def kernel(x, filters, table):
    xt = x.T
    out_t = _sc_gather(table, xt)
    return (out_t.transpose(2, 0, 1), filters)


# speedup vs baseline: 4.9876x; 1.0434x over previous
import functools

import jax
import jax.numpy as jnp
from jax import lax
from jax.experimental import pallas as pl
from jax.experimental.pallas import tpu as pltpu
from jax.experimental.pallas import tpu_sc as plsc

_BATCH = 4096
_FIELDS = 26
_EMB_DIM = 32
_VOCAB = 1000002
_NC, _NS = 2, 16
_NW = _NC * _NS
_NB = _BATCH // 128
_TILES = _FIELDS * _NB
_TPW = _TILES // _NW

_mesh = plsc.VectorSubcoreMesh(core_axis_name="c", subcore_axis_name="s")
_params = pltpu.CompilerParams(
    needs_layout_passes=False, use_tc_tiling_on_sc=False
)


@functools.partial(
    pl.kernel,
    out_type=jax.ShapeDtypeStruct((_FIELDS, _EMB_DIM, _BATCH), jnp.float32),
    mesh=_mesh,
    scratch_types=[
        pltpu.VMEM((128,), jnp.int32),
        pltpu.VMEM((128,), jnp.int32),
        pltpu.VMEM((128, _EMB_DIM), jnp.float32),
        pltpu.VMEM((128, _EMB_DIM), jnp.float32),
        pltpu.VMEM((_EMB_DIM, 128), jnp.float32),
        pltpu.VMEM((_EMB_DIM, 128), jnp.float32),
        pltpu.SemaphoreType.DMA,
        pltpu.SemaphoreType.DMA,
        pltpu.SemaphoreType.DMA,
        pltpu.SemaphoreType.DMA,
    ],
    compiler_params=_params,
)
def _sc_gather(table_hbm, xt_hbm, out_hbm, idxA, idxB, rowsA, rowsB,
               outA, outB, sgA, sgB, soA, soB):
    wid = lax.axis_index("s") * _NC + lax.axis_index("c")
    liota = lax.iota(jnp.int32, 16)
    idxs = (idxA, idxB)
    rows = (rowsA, rowsB)
    outs = (outA, outB)
    sgs = (sgA, sgB)
    sos = (soA, soB)

    def fb(t):
        g = wid * _TPW + t
        return g // _NB, g % _NB

    def fetch(t, p):
        f, b = fb(t)
        pltpu.sync_copy(xt_hbm.at[f, pl.ds(b * 128, 128)], idxs[p])
        pltpu.async_copy(table_hbm.at[idxs[p]], rows[p], sgs[p])

    def wait_gather(p):
        pltpu.make_async_copy(
            table_hbm.at[pl.ds(0, 128)], rows[p], sgs[p]
        ).wait()

    def dq(p):
        def body(q, c):
            lanes = liota + q * 16
            for dg in range(_EMB_DIM // 8):
                loads = [
                    plsc.load_gather(
                        rows[p],
                        [lanes, jnp.full((16,), 8 * dg + k, jnp.int32)],
                    )
                    for k in range(8)
                ]
                for k in range(8):
                    outs[p][8 * dg + k, pl.ds(q * 16, 16)] = loads[k]
            return c

        lax.fori_loop(0, 8, body, 0)

    def start_out(t, p):
        f, b = fb(t)
        pltpu.async_copy(
            outs[p], out_hbm.at[f, :, pl.ds(b * 128, 128)], sos[p]
        )

    def wait_out(p):
        pltpu.make_async_copy(
            outs[p], out_hbm.at[0, :, pl.ds(0, 128)], sos[p]
        ).wait()

    fetch(0, 0)

    def outer(s, carry):
        for p in (0, 1):
            t = 2 * s + p

            @pl.when(t + 1 < _TPW)
            def _prefetch():
                fetch(t + 1, 1 - p)

            wait_gather(p)
            dq(p)

            @pl.when(s >= 1)
            def _drain():
                wait_out(p)

            start_out(t, p)
        return carry

    lax.fori_loop(0, _TPW // 2, outer, 0)
    wait_out(0)
    wait_out(1)


def kernel(x, filters, table):
    xt = x.T
    out_t = _sc_gather(table, xt)
    return (out_t.transpose(2, 0, 1), filters)
